# Initial kernel scaffold; baseline (speedup 1.0000x reference)
import jax
import jax.numpy as jnp
from jax.experimental import pallas as pl


def kernel(pi, mask_parent, N):
    raise NotImplementedError("write your pallas kernel here")


# speedup vs baseline: 32.9288x; 32.9288x over previous
import functools

import jax
import jax.numpy as jnp
from jax import lax
from jax.experimental import pallas as pl
from jax.experimental.pallas import tpu as pltpu

_ROWS_PER_BLOCK = 8


def _mono_u32_from_f32(x):
    b = lax.bitcast_convert_type(x, jnp.uint32)
    m = jnp.where(b >= jnp.uint32(0x80000000),
                  jnp.uint32(0xFFFFFFFF), jnp.uint32(0x80000000))
    return b ^ m


def _mono_u32_from_i32(x):
    return lax.bitcast_convert_type(x, jnp.uint32) ^ jnp.uint32(0x80000000)


def _select_top_r(u, r, idx):
    def vbody(i, t):
        bit = (jnp.int32(31) - i).astype(jnp.uint32)
        cand = t | (jnp.uint32(1) << bit)
        cnt = jnp.sum((u >= cand).astype(jnp.int32), axis=-1, keepdims=True)
        return jnp.where(cnt >= r, cand, t)

    t = lax.fori_loop(0, 32, vbody, jnp.zeros(r.shape, jnp.uint32))

    gt = u > t
    c_gt = jnp.sum(gt.astype(jnp.int32), axis=-1, keepdims=True)
    need = r - c_gt
    tie = u == t

    def ibody(i, c):
        bit = jnp.int32(15) - i
        cand = c + (jnp.int32(1) << bit)
        cnt = jnp.sum((tie & (idx < cand)).astype(jnp.int32), axis=-1,
                      keepdims=True)
        return jnp.where(cnt <= need, cand, c)

    c = lax.fori_loop(0, 16, ibody, jnp.zeros(r.shape, jnp.int32))
    return (gt | (tie & (idx < c))).astype(jnp.int32)


def _quota_body(n_ref, pi_ref, mask_ref, out_ref):
    n_i = n_ref[0]
    n_f = n_i.astype(jnp.float32)
    r_rows, k = pi_ref.shape
    idx = lax.broadcasted_iota(jnp.int32, (r_rows, k), 1)

    valid = mask_ref[...] > 0.5
    pv = jnp.where(valid, pi_ref[...], 0.0)
    s = jnp.sum(pv, axis=-1, keepdims=True)
    s = jnp.clip(s, 1e-9, None)
    p = pv / s
    raw = p * n_f
    base = jnp.floor(raw).astype(jnp.int32)
    rem = n_i - jnp.sum(base, axis=-1, keepdims=True)
    frac = jnp.where(valid, raw - base.astype(jnp.float32),
                     jnp.float32(-1e9))

    add = _select_top_r(_mono_u32_from_f32(frac), rem, idx)
    n_k = jnp.where(valid, base + add, 0)
    out_ref[...] = n_k

    diff = n_i - jnp.sum(n_k, axis=-1, keepdims=True)

    @pl.when(jnp.any(diff != 0))
    def _():
        v = jnp.maximum(jnp.sum(valid.astype(jnp.int32), axis=-1,
                                keepdims=True), 1)
        dpos = jnp.maximum(diff, 0)
        key_pi = jnp.where(valid, p, -jnp.inf)
        inc = dpos // v + _select_top_r(_mono_u32_from_f32(key_pi),
                                       dpos % v, idx)
        dneg = jnp.maximum(-diff, 0)
        key_nk = jnp.where(valid, n_k, jnp.int32(-(2 ** 30)))
        dec = dneg // v + _select_top_r(_mono_u32_from_i32(key_nk),
                                       dneg % v, idx)
        n_k2 = n_k + jnp.where(valid, inc, 0)
        n_k2 = jnp.maximum(n_k2 - jnp.where(valid, dec, 0), 0)
        out_ref[...] = n_k2


@functools.partial(jax.jit, static_argnames=("interpret",))
def _quota_alloc(pi, mask_parent, n, interpret=False):
    b, k = pi.shape
    rb = _ROWS_PER_BLOCK
    n_arr = jnp.asarray(n, jnp.int32).reshape(1)
    return pl.pallas_call(
        _quota_body,
        grid=(b // rb,),
        in_specs=[
            pl.BlockSpec(memory_space=pltpu.SMEM),
            pl.BlockSpec((rb, k), lambda i: (i, 0)),
            pl.BlockSpec((rb, k), lambda i: (i, 0)),
        ],
        out_specs=pl.BlockSpec((rb, k), lambda i: (i, 0)),
        out_shape=jax.ShapeDtypeStruct((b, k), jnp.int32),
        compiler_params=pltpu.CompilerParams(
            dimension_semantics=("parallel",)),
        interpret=interpret,
    )(n_arr, pi, mask_parent)


def kernel(pi, mask_parent, N):
    return _quota_alloc(pi, mask_parent, N)


# speedup vs baseline: 46.3977x; 1.4090x over previous
import functools

import jax
import jax.numpy as jnp
from jax import lax
from jax.experimental import pallas as pl
from jax.experimental.pallas import tpu as pltpu

_ROWS_PER_BLOCK = 8


def _mono_u32_from_f32(x):
    b = lax.bitcast_convert_type(x, jnp.uint32)
    m = jnp.where(b >= jnp.uint32(0x80000000),
                  jnp.uint32(0xFFFFFFFF), jnp.uint32(0x80000000))
    return b ^ m


def _mono_u32_from_i32(x):
    return lax.bitcast_convert_type(x, jnp.uint32) ^ jnp.uint32(0x80000000)


def _select_top_r(u, r, idx):
    def vbody(i, t):
        bit = (jnp.int32(31) - i).astype(jnp.uint32)
        cand = t | (jnp.uint32(1) << bit)
        cnt = jnp.sum((u >= cand).astype(jnp.int32), axis=-1, keepdims=True)
        return jnp.where(cnt >= r, cand, t)

    t = lax.fori_loop(0, 32, vbody, jnp.zeros(r.shape, jnp.uint32))

    gt = u > t
    c_gt = jnp.sum(gt.astype(jnp.int32), axis=-1, keepdims=True)
    need = r - c_gt
    tie = u == t
    tie_cnt = jnp.sum(tie.astype(jnp.int32), axis=-1, keepdims=True)

    def _tie_search():
        def ibody(i, c):
            bit = jnp.int32(15) - i
            cand = c + (jnp.int32(1) << bit)
            cnt = jnp.sum((tie & (idx < cand)).astype(jnp.int32), axis=-1,
                          keepdims=True)
            return jnp.where(cnt <= need, cand, c)

        return lax.fori_loop(0, 16, ibody, jnp.zeros(r.shape, jnp.int32))

    def _tie_easy():
        return jnp.where(need <= 0, 0, jnp.int32(65536)) * jnp.ones(
            r.shape, jnp.int32)

    easy = (need <= 0) | (need >= tie_cnt)
    c = lax.cond(jnp.all(easy), _tie_easy, _tie_search)
    return (gt | (tie & (idx < c))).astype(jnp.int32)


def _quota_body(n_ref, pi_ref, mask_ref, out_ref):
    n_i = n_ref[0]
    n_f = n_i.astype(jnp.float32)
    r_rows, k = pi_ref.shape
    idx = lax.broadcasted_iota(jnp.int32, (r_rows, k), 1)

    valid = mask_ref[...] > 0.5
    pv = jnp.where(valid, pi_ref[...], 0.0)
    s = jnp.sum(pv, axis=-1, keepdims=True)
    s = jnp.clip(s, 1e-9, None)
    p = pv / s
    raw = p * n_f
    base = jnp.floor(raw).astype(jnp.int32)
    rem = n_i - jnp.sum(base, axis=-1, keepdims=True)
    frac = jnp.where(valid, raw - base.astype(jnp.float32),
                     jnp.float32(-1e9))

    add = _select_top_r(_mono_u32_from_f32(frac), rem, idx)
    n_k = jnp.where(valid, base + add, 0)
    out_ref[...] = n_k

    diff = n_i - jnp.sum(n_k, axis=-1, keepdims=True)

    @pl.when(jnp.any(diff != 0))
    def _():
        v = jnp.maximum(jnp.sum(valid.astype(jnp.int32), axis=-1,
                                keepdims=True), 1)
        dpos = jnp.maximum(diff, 0)
        key_pi = jnp.where(valid, p, -jnp.inf)
        inc = dpos // v + _select_top_r(_mono_u32_from_f32(key_pi),
                                       dpos % v, idx)
        dneg = jnp.maximum(-diff, 0)
        key_nk = jnp.where(valid, n_k, jnp.int32(-(2 ** 30)))
        dec = dneg // v + _select_top_r(_mono_u32_from_i32(key_nk),
                                       dneg % v, idx)
        n_k2 = n_k + jnp.where(valid, inc, 0)
        n_k2 = jnp.maximum(n_k2 - jnp.where(valid, dec, 0), 0)
        out_ref[...] = n_k2


@functools.partial(jax.jit, static_argnames=("interpret",))
def _quota_alloc(pi, mask_parent, n, interpret=False):
    b, k = pi.shape
    rb = _ROWS_PER_BLOCK
    n_arr = jnp.asarray(n, jnp.int32).reshape(1)
    return pl.pallas_call(
        _quota_body,
        grid=(b // rb,),
        in_specs=[
            pl.BlockSpec(memory_space=pltpu.SMEM),
            pl.BlockSpec((rb, k), lambda i: (i, 0)),
            pl.BlockSpec((rb, k), lambda i: (i, 0)),
        ],
        out_specs=pl.BlockSpec((rb, k), lambda i: (i, 0)),
        out_shape=jax.ShapeDtypeStruct((b, k), jnp.int32),
        compiler_params=pltpu.CompilerParams(
            dimension_semantics=("parallel",)),
        interpret=interpret,
    )(n_arr, pi, mask_parent)


def kernel(pi, mask_parent, N):
    return _quota_alloc(pi, mask_parent, N)
